# baseline (speedup 1.0000x reference)
import functools

import jax
import jax.numpy as jnp
from jax import lax
from jax.experimental import pallas as pl
from jax.experimental.pallas import tpu as pltpu
from jax.experimental.pallas import tpu_sc as plsc

_N_ROWS = 8192
_N_COLS = 4096
_LOG_N = 13
_BLOCK_COLS = 128
_SC_COLS = 512


def _sc_sort_rows(rows2, n=8192, log_n=13):
    r_rows = rows2.shape[0]
    nv = n // 16
    pi = [j + 4 for j in range(log_n - 4)] + [3, 2, 1, 0]
    mesh = plsc.VectorSubcoreMesh(core_axis_name="c", subcore_axis_name="s")

    @functools.partial(
        pl.kernel,
        mesh=mesh,
        out_type=jax.ShapeDtypeStruct((r_rows, n), jnp.float32),
        scratch_types=[pltpu.VMEM((n,), jnp.float32)],
        compiler_params=pltpu.CompilerParams(needs_layout_passes=False),
    )
    def k(in_hbm, out_hbm, buf):
        nc = 2
        cpw = r_rows // 32
        wid = lax.axis_index("s") * nc + lax.axis_index("c")
        iota16 = lax.iota(jnp.int32, 16)

        def boundary(bits):
            def body(v4, _):
                for u in range(4):
                    v = v4 * 4 + u
                    idx = v * 16 + iota16
                    x = buf[pl.ds(v * 16, 16)]
                    for b in bits:
                        x = x * (1 - 2 * ((idx >> b) & 1)).astype(
                            jnp.float32)
                    buf[pl.ds(v * 16, 16)] = x
                return 0
            lax.fori_loop(0, nv // 4, body, 0)

        def substage_pair(s):
            s16 = s // 16

            def body(p8, _):
                for u in range(8):
                    p = p8 * 8 + u
                    g = p // s16
                    t = p % s16
                    lo = (g * 2 * s16 + t) * 16
                    a = buf[pl.ds(lo, 16)]
                    b = buf[pl.ds(lo + s16 * 16, 16)]
                    buf[pl.ds(lo, 16)] = jnp.minimum(a, b)
                    buf[pl.ds(lo + s16 * 16, 16)] = jnp.maximum(a, b)
                return 0
            lax.fori_loop(0, nv // 16, body, 0)

        def substage_invreg(s):
            xor_idx = iota16 ^ s
            lane_hi = (iota16 & s) != 0

            def body(v4, _):
                for u in range(4):
                    base = (v4 * 4 + u) * 16
                    x = buf[pl.ds(base, 16)]
                    p = plsc.load_gather(buf, [base + xor_idx])
                    mn = jnp.minimum(x, p)
                    mx = jnp.maximum(x, p)
                    buf[pl.ds(base, 16)] = jnp.where(lane_hi, mx, mn)
                return 0
            lax.fori_loop(0, nv // 4, body, 0)

        def do_row(j, _):
            row = wid * cpw + j
            pltpu.sync_copy(in_hbm.at[row], buf)
            for kb in range(1, log_n + 1):
                bits = []
                if kb > 1:
                    bits.append(pi[kb - 1])
                if kb < log_n:
                    bits.append(pi[kb])
                boundary(bits)
                for jb in range(kb - 1, -1, -1):
                    s = 1 << pi[jb]
                    if s >= 16:
                        substage_pair(s)
                    else:
                        substage_invreg(s)
            pltpu.sync_copy(buf, out_hbm.at[row])
            return 0

        lax.fori_loop(0, cpw, do_row, 0)

    return k(rows2)


def _make_pi(log_n):
    return [j + 3 for j in range(log_n - 3)] + [2, 1, 0]


def _sign_table(n, log_n):
    pi = _make_pi(log_n)
    rows = jnp.arange(n, dtype=jnp.int32)[:, None]

    def sign(kb):
        return (1 - 2 * ((rows >> pi[kb]) & 1)).astype(jnp.float32)

    cols = []
    prev = None
    for kb in range(1, log_n + 1):
        cur = sign(kb) if kb < log_n else None
        if prev is None and cur is None:
            mult = jnp.ones((n, 1), jnp.float32)
        elif prev is None:
            mult = cur
        elif cur is None:
            mult = prev
        else:
            mult = prev * cur
        cols.append(mult)
        prev = cur
    while len(cols) < 13:
        cols.append(jnp.ones((n, 1), jnp.float32))
    for b in (2, 1, 0):
        cols.append(((rows >> b) & 1).astype(jnp.float32))
    return jnp.concatenate(cols, axis=1)


def _bitonic_sort_dim0(x, sg_ref, log_n):
    n, c = x.shape
    pi = _make_pi(log_n)
    ms = {4: sg_ref[:, 13:14] != 0,
          2: sg_ref[:, 14:15] != 0,
          1: sg_ref[:, 15:16] != 0}
    for kb in range(1, log_n + 1):
        x = x * sg_ref[:, kb - 1:kb]
        for jb in range(kb - 1, -1, -1):
            s = 1 << pi[jb]
            if s >= 8:
                g = n // (2 * s)
                x3 = x.reshape(g, 2 * s, c)
                mn = jnp.minimum(x3[:, :s, :], x3[:, s:, :])
                mx = jnp.maximum(x3[:, :s, :], x3[:, s:, :])
                x = jnp.concatenate([mn, mx], axis=1).reshape(n, c)
            else:
                mn_r = jnp.minimum(x, pltpu.roll(x, n - s, 0))
                mx_r = jnp.maximum(x, pltpu.roll(x, s, 0))
                x = jnp.where(ms[s], mx_r, mn_r)
    return x


def _fused_body(log_n, sg_ref, t_ref, f_ref, o_ref, acc_ref):
    i = pl.program_id(0)
    x = jnp.where(i % 2 == 0, t_ref[...], f_ref[...])
    srt = _bitonic_sort_dim0(x, sg_ref, log_n)

    @pl.when(i % 2 == 0)
    def _():
        acc_ref[...] = srt

    @pl.when(i == 1)
    def _():
        o_ref[...] = jnp.zeros_like(o_ref)

    @pl.when(i % 2 == 1)
    def _():
        d = acc_ref[...] - srt
        part = jnp.sum(
            jnp.maximum(d, 0.0) + jnp.log1p(jnp.exp(-jnp.abs(d))))
        o_ref[...] += jnp.full((1, 1), part, jnp.float32)


def _loss_body(t_ref, f_ref, o_ref):
    i = pl.program_id(0)
    d = t_ref[...] - f_ref[...]
    part = jnp.sum(jnp.maximum(d, 0.0) + jnp.log1p(jnp.exp(-jnp.abs(d))))

    @pl.when(i == 0)
    def _():
        o_ref[...] = jnp.zeros_like(o_ref)

    o_ref[...] += jnp.full((1, 1), part, jnp.float32)


def _tc_part(n_rows, total_cols, tc_cols, block_cols, log_n,
             interpret=False):
    nblk = tc_cols // block_cols
    off = (total_cols - tc_cols) // block_cols
    fused_call = pl.pallas_call(
        functools.partial(_fused_body, log_n),
        grid=(2 * nblk,),
        in_specs=[
            pl.BlockSpec((n_rows, 16), lambda i: (0, 0)),
            pl.BlockSpec((n_rows, block_cols),
                         lambda i, o=off: (0, o + i // 2)),
            pl.BlockSpec((n_rows, block_cols),
                         lambda i, o=off: (0, o + i // 2)),
        ],
        out_specs=pl.BlockSpec((1, 1), lambda i: (0, 0)),
        out_shape=jax.ShapeDtypeStruct((1, 1), jnp.float32),
        scratch_shapes=[
            pltpu.VMEM((n_rows, block_cols), jnp.float32),
        ],
        compiler_params=pltpu.CompilerParams(
            dimension_semantics=("arbitrary",)
        ),
        interpret=interpret,
    )

    def fn(true_data, fake_data):
        signs = _sign_table(n_rows, log_n)
        out = fused_call(signs, true_data, fake_data)
        return out[0, 0]

    return fn


def _make_loss_fn(n_rows, n_cols, block_cols, log_n, interpret=False):
    tc = _tc_part(n_rows, n_cols, n_cols, block_cols, log_n, interpret)

    def fn(true_data, fake_data):
        return tc(true_data, fake_data) / jnp.float32(n_rows * n_cols)

    return fn


def _sc_loss_call(sc_cols, n_rows, interpret=False):
    rblk = 64
    nblk = sc_cols // rblk
    return pl.pallas_call(
        _loss_body,
        grid=(nblk,),
        in_specs=[
            pl.BlockSpec((rblk, n_rows), lambda i: (i, 0)),
            pl.BlockSpec((rblk, n_rows), lambda i, nb=nblk: (i + nb, 0)),
        ],
        out_specs=pl.BlockSpec((1, 1), lambda i: (0, 0)),
        out_shape=jax.ShapeDtypeStruct((1, 1), jnp.float32),
        compiler_params=pltpu.CompilerParams(
            dimension_semantics=("arbitrary",)
        ),
        interpret=interpret,
    )


def kernel(true_data, fake_data):
    n, c = _N_ROWS, _N_COLS
    rows2 = jnp.concatenate(
        [true_data[:, :_SC_COLS].T, fake_data[:, :_SC_COLS].T], axis=0)
    sc_sorted = _sc_sort_rows(rows2, n, _LOG_N)
    tc_sum = _tc_part(n, c, c - _SC_COLS, _BLOCK_COLS, _LOG_N)(
        true_data, fake_data)
    sc_sum = _sc_loss_call(_SC_COLS, n)(sc_sorted, sc_sorted)[0, 0]
    return (tc_sum + sc_sum) / jnp.float32(n * c)


# speedup vs baseline: 1.0549x; 1.0549x over previous
import functools

import jax
import jax.numpy as jnp
from jax import lax
from jax.experimental import pallas as pl
from jax.experimental.pallas import tpu as pltpu
from jax.experimental.pallas import tpu_sc as plsc

_N_ROWS = 8192
_N_COLS = 4096
_LOG_N = 13
_BLOCK_COLS = 128
_SC_COLS = 384


def _sc_sort_rows(rows2, n=8192, log_n=13):
    r_rows = rows2.shape[0]
    nv = n // 16
    pi = [j + 4 for j in range(log_n - 4)] + [3, 2, 1, 0]
    mesh = plsc.VectorSubcoreMesh(core_axis_name="c", subcore_axis_name="s")

    @functools.partial(
        pl.kernel,
        mesh=mesh,
        out_type=jax.ShapeDtypeStruct((r_rows, n), jnp.float32),
        scratch_types=[pltpu.VMEM((n,), jnp.float32)],
        compiler_params=pltpu.CompilerParams(needs_layout_passes=False),
    )
    def k(in_hbm, out_hbm, buf):
        nc = 2
        cpw = r_rows // 32
        wid = lax.axis_index("s") * nc + lax.axis_index("c")
        iota16 = lax.iota(jnp.int32, 16)

        def boundary(bits):
            def body(v4, _):
                for u in range(4):
                    v = v4 * 4 + u
                    idx = v * 16 + iota16
                    x = buf[pl.ds(v * 16, 16)]
                    for b in bits:
                        x = x * (1 - 2 * ((idx >> b) & 1)).astype(
                            jnp.float32)
                    buf[pl.ds(v * 16, 16)] = x
                return 0
            lax.fori_loop(0, nv // 4, body, 0)

        def substage_pair(s):
            s16 = s // 16

            def body(p4, _):
                for u in range(4):
                    p = p4 * 4 + u
                    g = p // s16
                    t = p % s16
                    lo = (g * 2 * s16 + t) * 16
                    a = buf[pl.ds(lo, 16)]
                    b = buf[pl.ds(lo + s16 * 16, 16)]
                    buf[pl.ds(lo, 16)] = jnp.minimum(a, b)
                    buf[pl.ds(lo + s16 * 16, 16)] = jnp.maximum(a, b)
                return 0
            lax.fori_loop(0, nv // 8, body, 0)

        def substage_invreg(s):
            xor_idx = iota16 ^ s
            lane_hi = (iota16 & s) != 0

            def body(v4, _):
                for u in range(4):
                    base = (v4 * 4 + u) * 16
                    x = buf[pl.ds(base, 16)]
                    p = plsc.load_gather(buf, [base + xor_idx])
                    mn = jnp.minimum(x, p)
                    mx = jnp.maximum(x, p)
                    buf[pl.ds(base, 16)] = jnp.where(lane_hi, mx, mn)
                return 0
            lax.fori_loop(0, nv // 4, body, 0)

        def do_row(j, _):
            row = wid * cpw + j
            pltpu.sync_copy(in_hbm.at[row], buf)
            for kb in range(1, log_n + 1):
                bits = []
                if kb > 1:
                    bits.append(pi[kb - 1])
                if kb < log_n:
                    bits.append(pi[kb])
                boundary(bits)
                for jb in range(kb - 1, -1, -1):
                    s = 1 << pi[jb]
                    if s >= 16:
                        substage_pair(s)
                    else:
                        substage_invreg(s)
            pltpu.sync_copy(buf, out_hbm.at[row])
            return 0

        lax.fori_loop(0, cpw, do_row, 0)

    return k(rows2)


def _make_pi(log_n):
    return [j + 3 for j in range(log_n - 3)] + [2, 1, 0]


def _sign_table(n, log_n):
    pi = _make_pi(log_n)
    rows = jnp.arange(n, dtype=jnp.int32)[:, None]

    def sign(kb):
        return (1 - 2 * ((rows >> pi[kb]) & 1)).astype(jnp.float32)

    cols = []
    prev = None
    for kb in range(1, log_n + 1):
        cur = sign(kb) if kb < log_n else None
        if prev is None and cur is None:
            mult = jnp.ones((n, 1), jnp.float32)
        elif prev is None:
            mult = cur
        elif cur is None:
            mult = prev
        else:
            mult = prev * cur
        cols.append(mult)
        prev = cur
    while len(cols) < 13:
        cols.append(jnp.ones((n, 1), jnp.float32))
    for b in (2, 1, 0):
        cols.append(((rows >> b) & 1).astype(jnp.float32))
    return jnp.concatenate(cols, axis=1)


def _bitonic_sort_dim0(x, sg_ref, log_n):
    n, c = x.shape
    pi = _make_pi(log_n)
    ms = {4: sg_ref[:, 13:14] != 0,
          2: sg_ref[:, 14:15] != 0,
          1: sg_ref[:, 15:16] != 0}
    for kb in range(1, log_n + 1):
        x = x * sg_ref[:, kb - 1:kb]
        for jb in range(kb - 1, -1, -1):
            s = 1 << pi[jb]
            if s >= 8:
                g = n // (2 * s)
                x3 = x.reshape(g, 2 * s, c)
                mn = jnp.minimum(x3[:, :s, :], x3[:, s:, :])
                mx = jnp.maximum(x3[:, :s, :], x3[:, s:, :])
                x = jnp.concatenate([mn, mx], axis=1).reshape(n, c)
            else:
                mn_r = jnp.minimum(x, pltpu.roll(x, n - s, 0))
                mx_r = jnp.maximum(x, pltpu.roll(x, s, 0))
                x = jnp.where(ms[s], mx_r, mn_r)
    return x


def _fused_body(log_n, sg_ref, t_ref, f_ref, o_ref, acc_ref):
    i = pl.program_id(0)
    x = jnp.where(i % 2 == 0, t_ref[...], f_ref[...])
    srt = _bitonic_sort_dim0(x, sg_ref, log_n)

    @pl.when(i % 2 == 0)
    def _():
        acc_ref[...] = srt

    @pl.when(i == 1)
    def _():
        o_ref[...] = jnp.zeros_like(o_ref)

    @pl.when(i % 2 == 1)
    def _():
        d = acc_ref[...] - srt
        part = jnp.sum(
            jnp.maximum(d, 0.0) + jnp.log1p(jnp.exp(-jnp.abs(d))))
        o_ref[...] += jnp.full((1, 1), part, jnp.float32)


def _loss_body(t_ref, f_ref, o_ref):
    i = pl.program_id(0)
    d = t_ref[...] - f_ref[...]
    part = jnp.sum(jnp.maximum(d, 0.0) + jnp.log1p(jnp.exp(-jnp.abs(d))))

    @pl.when(i == 0)
    def _():
        o_ref[...] = jnp.zeros_like(o_ref)

    o_ref[...] += jnp.full((1, 1), part, jnp.float32)


def _tc_part(n_rows, total_cols, tc_cols, block_cols, log_n,
             interpret=False):
    nblk = tc_cols // block_cols
    off = (total_cols - tc_cols) // block_cols
    fused_call = pl.pallas_call(
        functools.partial(_fused_body, log_n),
        grid=(2 * nblk,),
        in_specs=[
            pl.BlockSpec((n_rows, 16), lambda i: (0, 0)),
            pl.BlockSpec((n_rows, block_cols),
                         lambda i, o=off: (0, o + i // 2)),
            pl.BlockSpec((n_rows, block_cols),
                         lambda i, o=off: (0, o + i // 2)),
        ],
        out_specs=pl.BlockSpec((1, 1), lambda i: (0, 0)),
        out_shape=jax.ShapeDtypeStruct((1, 1), jnp.float32),
        scratch_shapes=[
            pltpu.VMEM((n_rows, block_cols), jnp.float32),
        ],
        compiler_params=pltpu.CompilerParams(
            dimension_semantics=("arbitrary",)
        ),
        interpret=interpret,
    )

    def fn(true_data, fake_data):
        signs = _sign_table(n_rows, log_n)
        out = fused_call(signs, true_data, fake_data)
        return out[0, 0]

    return fn


def _make_loss_fn(n_rows, n_cols, block_cols, log_n, interpret=False):
    tc = _tc_part(n_rows, n_cols, n_cols, block_cols, log_n, interpret)

    def fn(true_data, fake_data):
        return tc(true_data, fake_data) / jnp.float32(n_rows * n_cols)

    return fn


def _sc_loss_call(sc_cols, n_rows, interpret=False):
    rblk = 64
    nblk = sc_cols // rblk
    return pl.pallas_call(
        _loss_body,
        grid=(nblk,),
        in_specs=[
            pl.BlockSpec((rblk, n_rows), lambda i: (i, 0)),
            pl.BlockSpec((rblk, n_rows), lambda i, nb=nblk: (i + nb, 0)),
        ],
        out_specs=pl.BlockSpec((1, 1), lambda i: (0, 0)),
        out_shape=jax.ShapeDtypeStruct((1, 1), jnp.float32),
        compiler_params=pltpu.CompilerParams(
            dimension_semantics=("arbitrary",)
        ),
        interpret=interpret,
    )


def kernel(true_data, fake_data):
    n, c = _N_ROWS, _N_COLS
    rows2 = jnp.concatenate(
        [true_data[:, :_SC_COLS].T, fake_data[:, :_SC_COLS].T], axis=0)
    sc_sorted = _sc_sort_rows(rows2, n, _LOG_N)
    tc_sum = _tc_part(n, c, c - _SC_COLS, _BLOCK_COLS, _LOG_N)(
        true_data, fake_data)
    sc_sum = _sc_loss_call(_SC_COLS, n)(sc_sorted, sc_sorted)[0, 0]
    return (tc_sum + sc_sum) / jnp.float32(n * c)
